# baseline (speedup 1.0000x reference)
import functools

import jax
import jax.numpy as jnp
from jax import lax
from jax.experimental import pallas as pl
from jax.experimental.pallas import tpu as pltpu
from jax.experimental.pallas import tpu_sc as plsc

NUM_NETS = 20000
NUM_PINS = 80000
NB = 512
INV_DX = 512.0 / 1000.0
INV_DY = 512.0 / 1000.0
EPS = 1.1920928955078125e-07
NW = 32
NETS_PER_W = NUM_NETS // NW
CH = 128
BIG = 3.0e38

_LANE = None


def _take16(v, idx):
    dnums = lax.GatherDimensionNumbers(
        offset_dims=(), collapsed_slice_dims=(0,), start_index_map=(0,))
    return lax.gather(
        v, idx[:, None], dimension_numbers=dnums, slice_sizes=(1,),
        mode=lax.GatherScatterMode.PROMISE_IN_BOUNDS)


def _sc_body(px_hbm, py_hbm, ns_hbm, fp_hbm, wt_hbm, zeros_hbm, out_hbm,
             bounds_v, wts_v, idx_v, gx_v, gy_v,
             xmin_v, xmax_v, ymin_v, ymax_v,
             midx_v, mval_v, px_sh, py_sh, d_sh):
    c = lax.axis_index("c")
    s = lax.axis_index("s")
    wid = c * 16 + s
    base = wid * NETS_PER_W
    r0 = lax.rem(wid, 8)
    lane = lax.iota(jnp.int32, 16)

    @pl.when(s == 0)
    def _():
        pltpu.sync_copy(px_hbm, px_sh)
    @pl.when(s == 1)
    def _():
        pltpu.sync_copy(py_hbm, py_sh)
    @pl.when(s == 2)
    def _():
        pltpu.sync_copy(zeros_hbm, d_sh)

    al = pl.multiple_of(base - r0, 8)
    pltpu.sync_copy(ns_hbm.at[pl.ds(al, 640)], bounds_v)
    pltpu.sync_copy(wt_hbm.at[pl.ds(al, 632)], wts_v)

    def _init(i, carry):
        o = i * 16
        xmin_v[pl.ds(o, 16)] = jnp.full((16,), BIG, jnp.float32)
        xmax_v[pl.ds(o, 16)] = jnp.full((16,), -BIG, jnp.float32)
        ymin_v[pl.ds(o, 16)] = jnp.full((16,), BIG, jnp.float32)
        ymax_v[pl.ds(o, 16)] = jnp.full((16,), -BIG, jnp.float32)
        return carry
    lax.fori_loop(0, 40, _init, 0)

    S = bounds_v[pl.ds(r0, 16)][0]
    E = bounds_v[pl.ds(r0 + NETS_PER_W - 15, 16)][15]
    a0 = (S // 16) * 16
    n_chunks = (E - a0 + (CH - 1)) // CH

    plsc.subcore_barrier()

    idm = [jnp.maximum(lane - t, 0) for t in (1, 2, 4, 8)]
    gem = [lane >= t for t in (1, 2, 4, 8)]
    idp = jnp.minimum(lane + 1, 15)

    def _chunk(t, carry):
        a = pl.multiple_of(a0 + t * CH, 8)
        pltpu.sync_copy(fp_hbm.at[pl.ds(a, CH)], idx_v)
        pltpu.sync_copy(px_sh.at[idx_v], gx_v)
        pltpu.sync_copy(py_sh.at[idx_v], gy_v)
        for kk in range(CH // 16):
            j16 = lane + (a + kk * 16)
            x16 = gx_v[pl.ds(kk * 16, 16)]
            y16 = gy_v[pl.ds(kk * 16, 16)]
            r = jnp.zeros((16,), jnp.int32)
            for st in (512, 256, 128, 64, 32, 16, 8, 4, 2, 1):
                cand = jnp.minimum(r + st, jnp.int32(NETS_PER_W))
                bv = plsc.load_gather(bounds_v, [cand + r0])
                r = jnp.where(bv <= j16, cand, r)
            valid = (j16 >= S) & (j16 < E)
            xmn = jnp.where(valid, x16, BIG)
            xmx = jnp.where(valid, x16, -BIG)
            ymn = jnp.where(valid, y16, BIG)
            ymx = jnp.where(valid, y16, -BIG)
            for t4 in range(4):
                ksh = _take16(r, idm[t4])
                upd = gem[t4] & (ksh == r)
                xmn = jnp.minimum(xmn, jnp.where(upd, _take16(xmn, idm[t4]), BIG))
                xmx = jnp.maximum(xmx, jnp.where(upd, _take16(xmx, idm[t4]), -BIG))
                ymn = jnp.minimum(ymn, jnp.where(upd, _take16(ymn, idm[t4]), BIG))
                ymx = jnp.maximum(ymx, jnp.where(upd, _take16(ymx, idm[t4]), -BIG))
            last = (r != _take16(r, idp)) | (lane == 15)
            smask = last & valid
            cur = plsc.load_gather(xmin_v, [r], mask=smask)
            plsc.store_scatter(xmin_v, [r], jnp.minimum(cur, xmn), mask=smask)
            cur = plsc.load_gather(xmax_v, [r], mask=smask)
            plsc.store_scatter(xmax_v, [r], jnp.maximum(cur, xmx), mask=smask)
            cur = plsc.load_gather(ymin_v, [r], mask=smask)
            plsc.store_scatter(ymin_v, [r], jnp.minimum(cur, ymn), mask=smask)
            cur = plsc.load_gather(ymax_v, [r], mask=smask)
            plsc.store_scatter(ymax_v, [r], jnp.maximum(cur, ymx), mask=smask)
        return carry
    lax.fori_loop(0, n_chunks, _chunk, 0)

    def _axis_masses(lo, hi, inv_d):
        l = lo * inv_d
        u = hi * inv_d
        wbin = (hi - lo) * inv_d
        ml = l.astype(jnp.int32)
        mu = u.astype(jnp.int32)
        fl = l - ml.astype(jnp.float32)
        fu = u - mu.astype(jnp.float32)
        g = (ml + 1).astype(jnp.float32) - l
        same = mu == ml
        adj = mu == ml + 1
        zero = jnp.zeros((16,), jnp.float32)
        v1 = jnp.where(same, wbin, g)
        v2 = jnp.where(same, -wbin, jnp.where(adj, fu - g, fl))
        v3 = jnp.where(same | adj, zero, fu - 1.0)
        v4 = jnp.where(same, zero, -fu)
        return (ml, ml + 1, mu, mu + 1), (v1, v2, v3, v4)

    def _group(gg, carry):
        o = gg * 16
        sv = bounds_v[pl.ds(r0 + o, 16)]
        ev = bounds_v[pl.ds(r0 + o + 1, 16)]
        inrange = (o + lane) < NETS_PER_W
        xmn = xmin_v[pl.ds(o, 16)]
        xmx = xmax_v[pl.ds(o, 16)]
        ymn = ymin_v[pl.ds(o, 16)]
        ymx = ymax_v[pl.ds(o, 16)]
        w_ = xmx - xmn
        h_ = ymx - ymn
        live = (ev > sv) & inrange & (w_ > 0) & (h_ > 0)
        wt16 = wts_v[pl.ds(r0 + o, 16)]
        coef = wt16 / ((w_ + EPS) * (h_ + EPS))
        coef = jnp.where(live, coef, 0.0)
        zf = jnp.zeros((16,), jnp.float32)
        xmn = jnp.where(live, xmn, zf)
        xmx = jnp.where(live, xmx, zf)
        ymn = jnp.where(live, ymn, zf)
        ymx = jnp.where(live, ymx, zf)
        mx, vx = _axis_masses(xmn, xmx, INV_DX)
        my, vy = _axis_masses(ymn, ymx, INV_DY)
        for ax in range(4):
            for ay in range(4):
                cc = ax * 4 + ay
                val = coef * vx[ax] * vy[ay]
                rr = mx[ax]
                ccol = my[ay]
                oob = (rr >= NB) | (ccol >= NB)
                val = jnp.where(oob, 0.0, val)
                fi = jnp.minimum(rr, NB - 1) * NB + jnp.minimum(ccol, NB - 1)
                row = 2 * gg + (cc // 8)
                col = (cc % 8) * 16
                midx_v[row, pl.ds(col, 16)] = fi
                mval_v[row, pl.ds(col, 16)] = val
        return carry
    lax.fori_loop(0, 40, _group, 0)

    def _scat(i, carry):
        pltpu.sync_copy(mval_v.at[i], d_sh.at[midx_v.at[i]], add=True)
        return carry
    lax.fori_loop(0, 80, _scat, 0)

    plsc.subcore_barrier()

    @pl.when(s == 0)
    def _():
        pltpu.sync_copy(d_sh, out_hbm.at[c])


def _sc_stage(px, py, ns_pad, fp_pad, wts, zeros):
    mesh = plsc.VectorSubcoreMesh(core_axis_name="c", subcore_axis_name="s")
    f = pl.kernel(
        _sc_body,
        out_type=jax.ShapeDtypeStruct((2, NB * NB), jnp.float32),
        mesh=mesh,
        compiler_params=pltpu.CompilerParams(needs_layout_passes=False),
        scratch_types=[
            pltpu.VMEM((640,), jnp.int32),
            pltpu.VMEM((632,), jnp.float32),
            pltpu.VMEM((CH,), jnp.int32),
            pltpu.VMEM((CH,), jnp.float32),
            pltpu.VMEM((CH,), jnp.float32),
            pltpu.VMEM((640,), jnp.float32),
            pltpu.VMEM((640,), jnp.float32),
            pltpu.VMEM((640,), jnp.float32),
            pltpu.VMEM((640,), jnp.float32),
            pltpu.VMEM((80, 128), jnp.int32),
            pltpu.VMEM((80, 128), jnp.float32),
            pltpu.VMEM_SHARED((NUM_PINS,), jnp.float32),
            pltpu.VMEM_SHARED((NUM_PINS,), jnp.float32),
            pltpu.VMEM_SHARED((NB * NB,), jnp.float32),
        ],
    )
    return f(px, py, ns_pad, fp_pad, wts, zeros)


def _tc_body(d_ref, o_ref):
    dd = d_ref[0] + d_ref[1]
    row = lax.broadcasted_iota(jnp.int32, (NB, NB), 0)
    col = lax.broadcasted_iota(jnp.int32, (NB, NB), 1)
    ltri = (row >= col).astype(jnp.float32)
    utri = (row <= col).astype(jnp.float32)
    cx = jnp.dot(ltri, dd, preferred_element_type=jnp.float32,
                 precision=lax.Precision.HIGHEST)
    o_ref[...] = jnp.dot(cx, utri, preferred_element_type=jnp.float32,
                         precision=lax.Precision.HIGHEST)


def _tc_stage(d2):
    return pl.pallas_call(
        _tc_body,
        out_shape=jax.ShapeDtypeStruct((NB, NB), jnp.float32),
    )(d2)


def kernel(pin_pos, netpin_start, flat_netpin, net_weights):
    px = pin_pos[:NUM_PINS]
    py = pin_pos[NUM_PINS:]
    ns_pad = jnp.concatenate(
        [netpin_start, jnp.full((7,), NUM_PINS, jnp.int32)])
    fp_pad = jnp.concatenate(
        [flat_netpin, jnp.zeros((640,), jnp.int32)])
    zeros = jnp.zeros((NB * NB,), jnp.float32)
    d2 = _sc_stage(px, py, ns_pad, fp_pad, net_weights, zeros)
    return _tc_stage(d2.reshape(2, NB, NB))


# speedup vs baseline: 1.0912x; 1.0912x over previous
import functools

import jax
import jax.numpy as jnp
from jax import lax
from jax.experimental import pallas as pl
from jax.experimental.pallas import tpu as pltpu
from jax.experimental.pallas import tpu_sc as plsc

NUM_NETS = 20000
NUM_PINS = 80000
NB = 512
INV_DX = 512.0 / 1000.0
INV_DY = 512.0 / 1000.0
EPS = 1.1920928955078125e-07
NW = 32
NETS_PER_W = NUM_NETS // NW
CH = 128
BIG = 3.0e38

_LANE = None


def _take16(v, idx):
    dnums = lax.GatherDimensionNumbers(
        offset_dims=(), collapsed_slice_dims=(0,), start_index_map=(0,))
    return lax.gather(
        v, idx[:, None], dimension_numbers=dnums, slice_sizes=(1,),
        mode=lax.GatherScatterMode.PROMISE_IN_BOUNDS)


def _sc_body(px_hbm, py_hbm, ns_hbm, fp_hbm, wt_hbm, zeros_hbm, out_hbm,
             bounds_v, wts_v, idx0_v, idx1_v, gx0_v, gx1_v, gy0_v, gy1_v,
             xmin_v, xmax_v, ymin_v, ymax_v,
             midx_v, mval_v, si0, si1, sg0, sg1, ssc, px_sh, py_sh, d_sh):
    c = lax.axis_index("c")
    s = lax.axis_index("s")
    wid = c * 16 + s
    base = wid * NETS_PER_W
    r0 = lax.rem(wid, 8)
    lane = lax.iota(jnp.int32, 16)

    @pl.when(s == 0)
    def _():
        pltpu.sync_copy(px_hbm, px_sh)
    @pl.when(s == 1)
    def _():
        pltpu.sync_copy(py_hbm, py_sh)
    @pl.when(s == 2)
    def _():
        pltpu.sync_copy(zeros_hbm, d_sh)

    al = pl.multiple_of(base - r0, 8)
    pltpu.sync_copy(ns_hbm.at[pl.ds(al, 640)], bounds_v)
    pltpu.sync_copy(wt_hbm.at[pl.ds(al, 632)], wts_v)

    def _init(i, carry):
        o = i * 16
        xmin_v[pl.ds(o, 16)] = jnp.full((16,), BIG, jnp.float32)
        xmax_v[pl.ds(o, 16)] = jnp.full((16,), -BIG, jnp.float32)
        ymin_v[pl.ds(o, 16)] = jnp.full((16,), BIG, jnp.float32)
        ymax_v[pl.ds(o, 16)] = jnp.full((16,), -BIG, jnp.float32)
        return carry
    lax.fori_loop(0, 40, _init, 0)

    S = bounds_v[pl.ds(r0, 16)][0]
    E = bounds_v[pl.ds(r0 + NETS_PER_W - 15, 16)][15]
    a0 = (S // 16) * 16
    n_chunks = (E - a0 + (CH - 1)) // CH

    plsc.subcore_barrier()

    idm = [jnp.maximum(lane - t, 0) for t in (1, 2, 4, 8)]
    gem = [lane >= t for t in (1, 2, 4, 8)]
    idp = jnp.minimum(lane + 1, 15)

    ixb = (idx0_v, idx1_v)
    gxb = (gx0_v, gx1_v)
    gyb = (gy0_v, gy1_v)
    sib = (si0, si1)
    sgb = (sg0, sg1)

    def _idx_src(t):
        a = pl.multiple_of(a0 + t * CH, 8)
        return fp_hbm.at[pl.ds(a, CH)]

    def _start_idx(t, b):
        pltpu.async_copy(_idx_src(t), ixb[b], sib[b])

    def _wait_idx(t, b):
        pltpu.make_async_copy(_idx_src(t), ixb[b], sib[b]).wait()

    def _start_g(b):
        pltpu.async_copy(px_sh.at[ixb[b]], gxb[b], sgb[b])
        pltpu.async_copy(py_sh.at[ixb[b]], gyb[b], sgb[b])

    def _wait_g(b):
        pltpu.make_async_copy(px_sh.at[ixb[b]], gxb[b], sgb[b]).wait()
        pltpu.make_async_copy(py_sh.at[ixb[b]], gyb[b], sgb[b]).wait()

    def _chunk(t, b):
        a = pl.multiple_of(a0 + t * CH, 8)
        gx_v = gxb[b]
        gy_v = gyb[b]
        for kk in range(CH // 16):
            j16 = lane + (a + kk * 16)
            x16 = gx_v[pl.ds(kk * 16, 16)]
            y16 = gy_v[pl.ds(kk * 16, 16)]
            r = jnp.zeros((16,), jnp.int32)
            for st in (512, 256, 128, 64, 32, 16, 8, 4, 2, 1):
                cand = jnp.minimum(r + st, jnp.int32(NETS_PER_W))
                bv = plsc.load_gather(bounds_v, [cand + r0])
                r = jnp.where(bv <= j16, cand, r)
            valid = (j16 >= S) & (j16 < E)
            xmn = jnp.where(valid, x16, BIG)
            xmx = jnp.where(valid, x16, -BIG)
            ymn = jnp.where(valid, y16, BIG)
            ymx = jnp.where(valid, y16, -BIG)
            for t4 in range(4):
                ksh = _take16(r, idm[t4])
                upd = gem[t4] & (ksh == r)
                xmn = jnp.minimum(xmn, jnp.where(upd, _take16(xmn, idm[t4]), BIG))
                xmx = jnp.maximum(xmx, jnp.where(upd, _take16(xmx, idm[t4]), -BIG))
                ymn = jnp.minimum(ymn, jnp.where(upd, _take16(ymn, idm[t4]), BIG))
                ymx = jnp.maximum(ymx, jnp.where(upd, _take16(ymx, idm[t4]), -BIG))
            last = (r != _take16(r, idp)) | (lane == 15)
            smask = last & valid
            cur = plsc.load_gather(xmin_v, [r], mask=smask)
            plsc.store_scatter(xmin_v, [r], jnp.minimum(cur, xmn), mask=smask)
            cur = plsc.load_gather(xmax_v, [r], mask=smask)
            plsc.store_scatter(xmax_v, [r], jnp.maximum(cur, xmx), mask=smask)
            cur = plsc.load_gather(ymin_v, [r], mask=smask)
            plsc.store_scatter(ymin_v, [r], jnp.minimum(cur, ymn), mask=smask)
            cur = plsc.load_gather(ymax_v, [r], mask=smask)
            plsc.store_scatter(ymax_v, [r], jnp.maximum(cur, ymx), mask=smask)

    nT = n_chunks

    @pl.when(nT > 0)
    def _():
        _start_idx(0, 0)

    @pl.when(nT > 1)
    def _():
        _start_idx(1, 1)

    @pl.when(nT > 0)
    def _():
        _wait_idx(0, 0)
        _start_g(0)

    def _iter(t, b):
        _wait_g(b)

        @pl.when(t + 1 < nT)
        def _():
            _wait_idx(t + 1, 1 - b)
            _start_g(1 - b)

        @pl.when(t + 2 < nT)
        def _():
            _start_idx(t + 2, b)

        _chunk(t, b)

    def _pipe(q, carry):
        t0 = q * 2

        @pl.when(t0 < nT)
        def _():
            _iter(t0, 0)

        @pl.when(t0 + 1 < nT)
        def _():
            _iter(t0 + 1, 1)

        return carry
    lax.fori_loop(0, (nT + 1) // 2, _pipe, 0)

    def _axis_masses(lo, hi, inv_d):
        l = lo * inv_d
        u = hi * inv_d
        wbin = (hi - lo) * inv_d
        ml = l.astype(jnp.int32)
        mu = u.astype(jnp.int32)
        fl = l - ml.astype(jnp.float32)
        fu = u - mu.astype(jnp.float32)
        g = (ml + 1).astype(jnp.float32) - l
        same = mu == ml
        adj = mu == ml + 1
        zero = jnp.zeros((16,), jnp.float32)
        v1 = jnp.where(same, wbin, g)
        v2 = jnp.where(same, -wbin, jnp.where(adj, fu - g, fl))
        v3 = jnp.where(same | adj, zero, fu - 1.0)
        v4 = jnp.where(same, zero, -fu)
        return (ml, ml + 1, mu, mu + 1), (v1, v2, v3, v4)

    def _group(gg, carry):
        o = gg * 16
        sv = bounds_v[pl.ds(r0 + o, 16)]
        ev = bounds_v[pl.ds(r0 + o + 1, 16)]
        inrange = (o + lane) < NETS_PER_W
        xmn = xmin_v[pl.ds(o, 16)]
        xmx = xmax_v[pl.ds(o, 16)]
        ymn = ymin_v[pl.ds(o, 16)]
        ymx = ymax_v[pl.ds(o, 16)]
        w_ = xmx - xmn
        h_ = ymx - ymn
        live = (ev > sv) & inrange & (w_ > 0) & (h_ > 0)
        wt16 = wts_v[pl.ds(r0 + o, 16)]
        coef = wt16 / ((w_ + EPS) * (h_ + EPS))
        coef = jnp.where(live, coef, 0.0)
        zf = jnp.zeros((16,), jnp.float32)
        xmn = jnp.where(live, xmn, zf)
        xmx = jnp.where(live, xmx, zf)
        ymn = jnp.where(live, ymn, zf)
        ymx = jnp.where(live, ymx, zf)
        mx, vx = _axis_masses(xmn, xmx, INV_DX)
        my, vy = _axis_masses(ymn, ymx, INV_DY)
        for ax in range(4):
            for ay in range(4):
                cc = ax * 4 + ay
                val = coef * vx[ax] * vy[ay]
                rr = mx[ax]
                ccol = my[ay]
                oob = (rr >= NB) | (ccol >= NB)
                val = jnp.where(oob, 0.0, val)
                fi = jnp.minimum(rr, NB - 1) * NB + jnp.minimum(ccol, NB - 1)
                row = 2 * gg + (cc // 8)
                col = (cc % 8) * 16
                midx_v[row, pl.ds(col, 16)] = fi
                mval_v[row, pl.ds(col, 16)] = val
        return carry
    lax.fori_loop(0, 40, _group, 0)

    def _scat(q, carry):
        i0 = q * 8
        for j in range(8):
            pltpu.async_copy(mval_v.at[i0 + j], d_sh.at[midx_v.at[i0 + j]],
                             ssc, add=True)
        for j in range(8):
            pltpu.make_async_copy(mval_v.at[i0 + j],
                                  d_sh.at[midx_v.at[i0 + j]], ssc).wait()
        return carry
    lax.fori_loop(0, 10, _scat, 0)

    plsc.subcore_barrier()

    @pl.when(s == 0)
    def _():
        pltpu.sync_copy(d_sh, out_hbm.at[c])


def _sc_stage(px, py, ns_pad, fp_pad, wts, zeros):
    mesh = plsc.VectorSubcoreMesh(core_axis_name="c", subcore_axis_name="s")
    f = pl.kernel(
        _sc_body,
        out_type=jax.ShapeDtypeStruct((2, NB * NB), jnp.float32),
        mesh=mesh,
        compiler_params=pltpu.CompilerParams(needs_layout_passes=False),
        scratch_types=[
            pltpu.VMEM((640,), jnp.int32),
            pltpu.VMEM((632,), jnp.float32),
            pltpu.VMEM((CH,), jnp.int32),
            pltpu.VMEM((CH,), jnp.int32),
            pltpu.VMEM((CH,), jnp.float32),
            pltpu.VMEM((CH,), jnp.float32),
            pltpu.VMEM((CH,), jnp.float32),
            pltpu.VMEM((CH,), jnp.float32),
            pltpu.VMEM((640,), jnp.float32),
            pltpu.VMEM((640,), jnp.float32),
            pltpu.VMEM((640,), jnp.float32),
            pltpu.VMEM((640,), jnp.float32),
            pltpu.VMEM((80, 128), jnp.int32),
            pltpu.VMEM((80, 128), jnp.float32),
            pltpu.SemaphoreType.DMA,
            pltpu.SemaphoreType.DMA,
            pltpu.SemaphoreType.DMA,
            pltpu.SemaphoreType.DMA,
            pltpu.SemaphoreType.DMA,
            pltpu.VMEM_SHARED((NUM_PINS,), jnp.float32),
            pltpu.VMEM_SHARED((NUM_PINS,), jnp.float32),
            pltpu.VMEM_SHARED((NB * NB,), jnp.float32),
        ],
    )
    return f(px, py, ns_pad, fp_pad, wts, zeros)


def _tc_body(d_ref, o_ref):
    dd = d_ref[0] + d_ref[1]
    row = lax.broadcasted_iota(jnp.int32, (NB, NB), 0)
    col = lax.broadcasted_iota(jnp.int32, (NB, NB), 1)
    ltri = (row >= col).astype(jnp.float32)
    utri = (row <= col).astype(jnp.float32)
    cx = jnp.dot(ltri, dd, preferred_element_type=jnp.float32,
                 precision=lax.Precision.HIGHEST)
    o_ref[...] = jnp.dot(cx, utri, preferred_element_type=jnp.float32,
                         precision=lax.Precision.HIGHEST)


def _tc_stage(d2):
    return pl.pallas_call(
        _tc_body,
        out_shape=jax.ShapeDtypeStruct((NB, NB), jnp.float32),
    )(d2)


def kernel(pin_pos, netpin_start, flat_netpin, net_weights):
    px = pin_pos[:NUM_PINS]
    py = pin_pos[NUM_PINS:]
    ns_pad = jnp.concatenate(
        [netpin_start, jnp.full((7,), NUM_PINS, jnp.int32)])
    fp_pad = jnp.concatenate(
        [flat_netpin, jnp.zeros((640,), jnp.int32)])
    zeros = jnp.zeros((NB * NB,), jnp.float32)
    d2 = _sc_stage(px, py, ns_pad, fp_pad, net_weights, zeros)
    return _tc_stage(d2.reshape(2, NB, NB))
